# baseline (speedup 1.0000x reference)
import functools

import jax
import jax.numpy as jnp
from jax import lax
from jax.experimental import pallas as pl
from jax.experimental.pallas import tpu as pltpu
from jax.experimental.pallas import tpu_sc as plsc

NC = 2
NS = 16
NW = NC * NS

IDS_MINOR = 128
CHUNK = 512
NBUF = 2


def _sc_gather_diff(features2d, ids2d, num_tuples, const_dim):
    tw = num_tuples // NW
    nch = tw // CHUNK
    gpc = 2 * CHUNK // IDS_MINOR
    rows_per_tile = 2 * tw // IDS_MINOR
    half = const_dim // 2

    mesh = plsc.VectorSubcoreMesh(
        core_axis_name="c", subcore_axis_name="s",
        num_cores=NC, num_subcores=NS)

    @functools.partial(
        pl.kernel,
        out_type=jax.ShapeDtypeStruct((num_tuples, const_dim), jnp.float32),
        mesh=mesh,
        scratch_types=[
            pltpu.VMEM((rows_per_tile, IDS_MINOR), jnp.int32),
            pltpu.VMEM((NBUF, 2 * CHUNK, const_dim), jnp.float32),
            pltpu.VMEM((CHUNK, const_dim), jnp.float32),
            pltpu.SemaphoreType.DMA,
            pltpu.SemaphoreType.DMA,
        ],
        compiler_params=pltpu.CompilerParams(use_tc_tiling_on_sc=False),
    )
    def sc_fn(feat_hbm, ids_hbm, out_hbm, idx_v, rows_v, out_v, sem0, sem1):
        wid = lax.axis_index("s") * NC + lax.axis_index("c")
        sems = (sem0, sem1)

        pltpu.sync_copy(ids_hbm.at[pl.ds(wid * rows_per_tile, rows_per_tile)],
                        idx_v)

        def fire(ch, slot):
            descs = []
            for j in range(gpc):
                descs.append(pltpu.async_copy(
                    feat_hbm.at[idx_v.at[ch * gpc + j]],
                    rows_v.at[slot].at[pl.ds(j * IDS_MINOR, IDS_MINOR)],
                    sems[slot]))
            return descs

        pending = {0: fire(0, 0), 1: None}
        for ch in range(nch):
            slot = ch % NBUF
            if ch + 1 < nch:
                pending[(ch + 1) % NBUF] = fire(ch + 1, (ch + 1) % NBUF)
            for d in pending[slot]:
                d.wait()

            def sub_body(i, _):
                h0 = rows_v[slot, 2 * i, pl.ds(0, 16)]
                t0 = rows_v[slot, 2 * i + 1, pl.ds(0, 16)]
                out_v[i, pl.ds(0, 16)] = h0 - t0
                h1 = rows_v[slot, 2 * i, pl.ds(half, 16)]
                t1 = rows_v[slot, 2 * i + 1, pl.ds(half, 16)]
                out_v[i, pl.ds(half, 16)] = h1 - t1
                return 0

            lax.fori_loop(0, CHUNK, sub_body, 0, unroll=4)
            base = wid * tw + ch * CHUNK
            pltpu.sync_copy(out_v, out_hbm.at[pl.ds(base, CHUNK)])

    return sc_fn(features2d, ids2d)


def _tc_project_normalize(trans, r2d, W, b2d, num_tuples, const_dim, atom_dim):
    bt = 8192
    grid = (num_tuples // bt,)

    def tc_body(trans_ref, r_ref, w_ref, b_ref, out_ref):
        c = jnp.dot(r_ref[...], w_ref[...],
                    preferred_element_type=jnp.float32) + b_ref[...]
        emb = jnp.dot(trans_ref[...], w_ref[...],
                      preferred_element_type=jnp.float32) + c
        ss = jnp.sum(emb * emb, axis=-1, keepdims=True)
        out_ref[...] = emb * lax.rsqrt(jnp.maximum(ss, 1e-12))

    return pl.pallas_call(
        tc_body,
        grid=grid,
        in_specs=[
            pl.BlockSpec((bt, const_dim), lambda i: (i, 0)),
            pl.BlockSpec((1, const_dim), lambda i: (0, 0)),
            pl.BlockSpec((const_dim, atom_dim), lambda i: (0, 0)),
            pl.BlockSpec((1, atom_dim), lambda i: (0, 0)),
        ],
        out_specs=pl.BlockSpec((bt, atom_dim), lambda i: (i, 0)),
        out_shape=jax.ShapeDtypeStruct((num_tuples, atom_dim), jnp.float32),
        compiler_params=pltpu.CompilerParams(
            dimension_semantics=("arbitrary",)),
    )(trans, r2d, W, b2d)


def kernel(features, ids, r, W, b):
    _, num_constants, const_dim = features.shape
    num_tuples = ids.shape[0]
    atom_dim = W.shape[1]

    features2d = features.reshape(num_constants, const_dim)
    ids2d = ids.reshape(2 * num_tuples // IDS_MINOR, IDS_MINOR)

    trans = _sc_gather_diff(features2d, ids2d, num_tuples, const_dim)

    out = _tc_project_normalize(trans, r.reshape(1, const_dim), W,
                                b.reshape(1, atom_dim),
                                num_tuples, const_dim, atom_dim)
    return out.reshape(1, num_tuples, atom_dim)


# speedup vs baseline: 1.1026x; 1.1026x over previous
import functools

import jax
import jax.numpy as jnp
from jax import lax
from jax.experimental import pallas as pl
from jax.experimental.pallas import tpu as pltpu
from jax.experimental.pallas import tpu_sc as plsc

NC = 2
NS = 16
NW = NC * NS

IDS_MINOR = 128
CHUNK = 512
NBUF = 2


def _tc_project(featT, W, num_constants, const_dim, atom_dim):
    bn = 8192
    grid = (pl.cdiv(num_constants, bn),)

    def body(featT_ref, w_ref, g_ref):
        g_ref[...] = lax.dot_general(
            featT_ref[...], w_ref[...], (((0,), (0,)), ((), ())),
            preferred_element_type=jnp.float32)

    return pl.pallas_call(
        body,
        grid=grid,
        in_specs=[
            pl.BlockSpec((const_dim, bn), lambda i: (0, i)),
            pl.BlockSpec((const_dim, atom_dim), lambda i: (0, 0)),
        ],
        out_specs=pl.BlockSpec((bn, atom_dim), lambda i: (i, 0)),
        out_shape=jax.ShapeDtypeStruct((num_constants, atom_dim), jnp.float32),
        compiler_params=pltpu.CompilerParams(
            dimension_semantics=("arbitrary",)),
    )(featT, W)


def _sc_gather_diff(table, idsv, num_tuples, dim):
    tw = num_tuples // NW
    nch = tw // CHUNK
    gpc = 2 * CHUNK // IDS_MINOR
    ppc = CHUNK // IDS_MINOR
    rows_per_tile = 2 * tw // IDS_MINOR

    mesh = plsc.VectorSubcoreMesh(
        core_axis_name="c", subcore_axis_name="s",
        num_cores=NC, num_subcores=NS)

    @functools.partial(
        pl.kernel,
        out_type=jax.ShapeDtypeStruct((num_tuples, dim), jnp.float32),
        mesh=mesh,
        scratch_types=[
            pltpu.VMEM((rows_per_tile, IDS_MINOR), jnp.int32),
            pltpu.VMEM((NBUF, 2 * CHUNK, dim), jnp.float32),
            pltpu.VMEM((CHUNK, dim), jnp.float32),
            pltpu.SemaphoreType.DMA,
            pltpu.SemaphoreType.DMA,
        ],
        compiler_params=pltpu.CompilerParams(use_tc_tiling_on_sc=False),
    )
    def sc_fn(table_hbm, ids_hbm, out_hbm, idx_v, rows_v, out_v, sem0, sem1):
        wid = lax.axis_index("s") * NC + lax.axis_index("c")
        sems = (sem0, sem1)

        pltpu.sync_copy(ids_hbm.at[pl.ds(wid * rows_per_tile, rows_per_tile)],
                        idx_v)

        def fire(ch, slot):
            descs = []
            for j in range(gpc):
                descs.append(pltpu.async_copy(
                    table_hbm.at[idx_v.at[ch * gpc + j]],
                    rows_v.at[slot].at[pl.ds(j * IDS_MINOR, IDS_MINOR)],
                    sems[slot]))
            return descs

        pending = {0: fire(0, 0), 1: None}
        half = dim // 2
        for ch in range(nch):
            slot = ch % NBUF
            if ch + 1 < nch:
                pending[(ch + 1) % NBUF] = fire(ch + 1, (ch + 1) % NBUF)
            for d in pending[slot]:
                d.wait()

            for p in range(ppc):
                hrow = 2 * p * IDS_MINOR
                trow = hrow + IDS_MINOR

                def sub_body(c, _, hrow=hrow, trow=trow, p=p):
                    h0 = rows_v[slot, hrow + c, pl.ds(0, 16)]
                    t0 = rows_v[slot, trow + c, pl.ds(0, 16)]
                    out_v[p * IDS_MINOR + c, pl.ds(0, 16)] = h0 - t0
                    h1 = rows_v[slot, hrow + c, pl.ds(half, 16)]
                    t1 = rows_v[slot, trow + c, pl.ds(half, 16)]
                    out_v[p * IDS_MINOR + c, pl.ds(half, 16)] = h1 - t1
                    return 0

                lax.fori_loop(0, IDS_MINOR, sub_body, 0, unroll=4)
            base = wid * tw + ch * CHUNK
            pltpu.sync_copy(out_v, out_hbm.at[pl.ds(base, CHUNK)])

    return sc_fn(table, idsv)


def _tc_norm_t(trans, r2d, W, b2d, num_tuples, const_dim, atom_dim):
    bt = 8192
    grid = (num_tuples // bt,)
    eye = jnp.eye(atom_dim, dtype=jnp.float32)

    def body(trans_ref, r_ref, w_ref, b_ref, eye_ref, out_ref):
        c = jnp.dot(r_ref[...], w_ref[...],
                    preferred_element_type=jnp.float32) + b_ref[...]
        emb = trans_ref[...] + c
        ss = jnp.sum(emb * emb, axis=-1, keepdims=True)
        y = emb * lax.rsqrt(jnp.maximum(ss, 1e-12))
        out_ref[...] = lax.dot_general(
            eye_ref[...], y, (((1,), (1,)), ((), ())),
            preferred_element_type=jnp.float32)

    return pl.pallas_call(
        body,
        grid=grid,
        in_specs=[
            pl.BlockSpec((bt, const_dim), lambda i: (i, 0)),
            pl.BlockSpec((1, const_dim), lambda i: (0, 0)),
            pl.BlockSpec((const_dim, atom_dim), lambda i: (0, 0)),
            pl.BlockSpec((1, atom_dim), lambda i: (0, 0)),
            pl.BlockSpec((atom_dim, atom_dim), lambda i: (0, 0)),
        ],
        out_specs=pl.BlockSpec((atom_dim, bt), lambda i: (0, i)),
        out_shape=jax.ShapeDtypeStruct((atom_dim, num_tuples), jnp.float32),
        compiler_params=pltpu.CompilerParams(
            dimension_semantics=("arbitrary",)),
    )(trans, r2d, W, b2d, eye)


def kernel(features, ids, r, W, b):
    _, num_constants, const_dim = features.shape
    num_tuples = ids.shape[0]
    atom_dim = W.shape[1]

    featT = features[0].T
    idsv = (ids.reshape(num_tuples // IDS_MINOR, IDS_MINOR, 2)
               .transpose(0, 2, 1)
               .reshape(2 * num_tuples // IDS_MINOR, IDS_MINOR))

    G = _tc_project(featT, W, num_constants, const_dim, atom_dim)
    trans = _sc_gather_diff(G, idsv, num_tuples, atom_dim)
    outT = _tc_norm_t(trans, r.reshape(1, const_dim), W,
                      b.reshape(1, atom_dim), num_tuples, const_dim, atom_dim)
    return outT.T.reshape(1, num_tuples, atom_dim)
